# Initial kernel scaffold; baseline (speedup 1.0000x reference)
import jax
import jax.numpy as jnp
from jax.experimental import pallas as pl


def kernel(ray_origins, ray_directions, density_grid, sh_grid):
    raise NotImplementedError("write your pallas kernel here")


# speedup vs baseline: 7.2627x; 7.2627x over previous
import functools

import jax
import jax.numpy as jnp
from jax import lax
from jax.experimental import pallas as pl
from jax.experimental.pallas import tpu as pltpu
from jax.experimental.pallas import tpu_sc as plsc

_GRID = 128
_NC = 9
_NS = 64
_NEAR = 0.1
_FAR = 4.0
_NRAYS = 4096
_BLK_R = 16
_NCORN = 8
_NPTS = _NRAYS * _NS
_B = _NPTS * _NCORN
_NW = 32
_CHUNK = 2048


def _prep_call(o, d):
    grid_n = _NRAYS // _BLK_R

    def body(o_ref, d_ref, idx_ref, w_ref, b_ref):
        o_b = o_ref[...]
        d_b = d_ref[...]
        tt = _NEAR + lax.broadcasted_iota(
            jnp.int32, (_BLK_R, _NS), 1).astype(jnp.float32) * (
            (_FAR - _NEAR) / (_NS - 1))
        fs, cs, fr = [], [], []
        for a in range(3):
            oa = o_b[:, a:a + 1]
            da = d_b[:, a:a + 1]
            pa = (oa + da * tt + 1.0) * (0.5 * (_GRID - 1))
            pc = jnp.clip(pa, 0.0, float(_GRID - 1))
            fa = jnp.floor(pc)
            fr.append(pc - fa)
            fi = fa.astype(jnp.int32)
            fs.append(fi)
            cs.append(jnp.minimum(fi + 1, _GRID - 1))
        idxs, ws = [], []
        for dx in (0, 1):
            X = fs[0] if dx == 0 else cs[0]
            wx = (1.0 - fr[0]) if dx == 0 else fr[0]
            for dy in (0, 1):
                Y = fs[1] if dy == 0 else cs[1]
                wy = (1.0 - fr[1]) if dy == 0 else fr[1]
                for dz in (0, 1):
                    Z = fs[2] if dz == 0 else cs[2]
                    wz = (1.0 - fr[2]) if dz == 0 else fr[2]
                    idxs.append((X * _GRID + Y) * _GRID + Z)
                    ws.append(wx * wy * wz)
        idx_ref[...] = jnp.stack(idxs, axis=0)
        w_ref[...] = jnp.stack(ws, axis=0)
        x, y, z = d_b[:, 0], d_b[:, 1], d_b[:, 2]
        c0 = 0.28209479177387814
        c1 = 0.4886025119029199
        b_ref[...] = jnp.stack([
            jnp.full_like(x, c0),
            -c1 * y,
            c1 * z,
            -c1 * x,
            1.0925484305920792 * x * y,
            -1.0925484305920792 * y * z,
            0.31539156525252005 * (2.0 * z * z - x * x - y * y),
            -1.0925484305920792 * x * z,
            0.5462742152960396 * (x * x - y * y),
        ], axis=-1)

    return pl.pallas_call(
        body,
        grid=(grid_n,),
        in_specs=[
            pl.BlockSpec((_BLK_R, 3), lambda i: (i, 0)),
            pl.BlockSpec((_BLK_R, 3), lambda i: (i, 0)),
        ],
        out_specs=[
            pl.BlockSpec((_NCORN, _BLK_R, _NS), lambda i: (0, i, 0)),
            pl.BlockSpec((_NCORN, _BLK_R, _NS), lambda i: (0, i, 0)),
            pl.BlockSpec((_BLK_R, _NC), lambda i: (i, 0)),
        ],
        out_shape=[
            jax.ShapeDtypeStruct((_NCORN, _NRAYS, _NS), jnp.int32),
            jax.ShapeDtypeStruct((_NCORN, _NRAYS, _NS), jnp.float32),
            jax.ShapeDtypeStruct((_NRAYS, _NC), jnp.float32),
        ],
    )(o, d)


def _pack_call(sh_tab, den_tab):
    blk = 8192
    grid_n = (_GRID ** 3) // blk

    def body(sh_ref, den_ref, out_ref):
        sh_b = sh_ref[...]
        den_b = den_ref[...]
        pad = jnp.zeros((blk, 4), jnp.float32)
        out_ref[...] = jnp.concatenate([sh_b, den_b, pad], axis=1)

    return pl.pallas_call(
        body,
        grid=(grid_n,),
        in_specs=[
            pl.BlockSpec((blk, 3 * _NC), lambda i: (i, 0)),
            pl.BlockSpec((blk, 1), lambda i: (i, 0)),
        ],
        out_specs=pl.BlockSpec((blk, 32), lambda i: (i, 0)),
        out_shape=jax.ShapeDtypeStruct((_GRID ** 3, 32), jnp.float32),
    )(sh_tab, den_tab)


def _sc_gather(idx_flat, tab):
    b_per_w = _B // _NW
    n_chunks = b_per_w // _CHUNK
    mesh = plsc.VectorSubcoreMesh(core_axis_name="c", subcore_axis_name="s")

    @functools.partial(
        pl.kernel, mesh=mesh,
        compiler_params=pltpu.CompilerParams(use_tc_tiling_on_sc=False),
        out_type=jax.ShapeDtypeStruct((_B, 32), jnp.float32),
        scratch_types=[
            pltpu.VMEM((_CHUNK,), jnp.int32),
            pltpu.VMEM((_CHUNK, 32), jnp.float32),
            pltpu.SemaphoreType.DMA,
        ],
    )
    def k(idx_hbm, tab_hbm, rows_out, idx_v, rows_v, sem1):
        wid = lax.axis_index("s") * 2 + lax.axis_index("c")
        base = wid * b_per_w

        def step(ci, carry):
            off = pl.multiple_of(base + ci * _CHUNK, _CHUNK)
            pltpu.sync_copy(idx_hbm.at[pl.ds(off, _CHUNK)], idx_v)
            pltpu.async_copy(tab_hbm.at[idx_v], rows_v, sem1).wait()
            pltpu.sync_copy(rows_v, rows_out.at[pl.ds(off, _CHUNK)])
            return carry

        lax.fori_loop(0, n_chunks, step, 0)

    return k(idx_flat, tab)


def _combine_call(rows, w8, basis):
    grid_n = _NRAYS // _BLK_R
    delta = (_FAR - _NEAR) / _NS

    def body(rows_ref, w_ref, b_ref, out_ref):
        rows_b = rows_ref[...]
        w_b = w_ref[...]
        b_b = b_ref[...]
        sh = jnp.sum(rows_b * w_b[..., None], axis=0)
        sigma = jnp.maximum(sh[:, :, 27], 0.0)
        alpha = 1.0 - jnp.exp(-sigma * delta)
        logs = jnp.log(1.0 - alpha + 1e-10)
        r_i = lax.broadcasted_iota(jnp.int32, (_NS, _NS), 0)
        c_i = lax.broadcasted_iota(jnp.int32, (_NS, _NS), 1)
        tri = (r_i < c_i).astype(jnp.float32)
        trans = jnp.exp(jnp.dot(logs, tri,
                                precision=jax.lax.Precision.HIGHEST,
                                preferred_element_type=jnp.float32))
        wts = alpha * trans
        outs = []
        for c in range(3):
            pre = jnp.sum(sh[:, :, _NC * c:_NC * (c + 1)] * b_b[:, None, :],
                          axis=-1)
            rgb_c = 1.0 / (1.0 + jnp.exp(-pre))
            outs.append(jnp.sum(wts * rgb_c, axis=-1))
        out_ref[...] = jnp.stack(outs, axis=-1)

    return pl.pallas_call(
        body,
        grid=(grid_n,),
        in_specs=[
            pl.BlockSpec((_NCORN, _BLK_R, _NS, 32), lambda i: (0, i, 0, 0)),
            pl.BlockSpec((_NCORN, _BLK_R, _NS), lambda i: (0, i, 0)),
            pl.BlockSpec((_BLK_R, _NC), lambda i: (i, 0)),
        ],
        out_specs=pl.BlockSpec((_BLK_R, 3), lambda i: (i, 0)),
        out_shape=jax.ShapeDtypeStruct((_NRAYS, 3), jnp.float32),
    )(rows, w8, basis)


def kernel(ray_origins, ray_directions, density_grid, sh_grid):
    sh_tab = sh_grid.reshape(_GRID ** 3, 3 * _NC)
    den_tab = density_grid.reshape(_GRID ** 3, 1)
    tab = _pack_call(sh_tab, den_tab)
    idx8, w8, basis = _prep_call(ray_origins, ray_directions)
    rows = _sc_gather(idx8.reshape(-1), tab)
    return _combine_call(
        rows.reshape(_NCORN, _NRAYS, _NS, 32),
        w8, basis)
